# baseline (speedup 1.0000x reference)
import jax
import jax.numpy as jnp
from jax.experimental import pallas as pl
from jax.experimental.pallas import tpu as pltpu

_B = 64
_SLOTS = 4096
_DK = 128
_DV = 128
_BPB = 4


def _recall_body(ptr_ref, k_ref, v_ref, c_ref, keys_ref, vals_ref,
                 vhat_ref, alpha_ref):
    g = pl.program_id(0)
    col = jax.lax.broadcasted_iota(jnp.int32, (1, _SLOTS), 1)
    for i in range(_BPB):
        b = g * _BPB + i
        p = ptr_ref[b]
        keys = keys_ref[i]
        vals = vals_ref[i]
        c_row = c_ref[pl.ds(b, 1), :]

        sims = jax.lax.dot_general(
            c_row, keys, (((1,), (1,)), ((), ())),
            preferred_element_type=jnp.float32)

        d_new = jnp.sum(k_ref[pl.ds(b, 1), :] * c_row)
        is_p = col == p
        sims = jnp.where(is_p, d_new, sims)

        m = jnp.max(sims)
        e = jnp.exp(sims - m)
        s = jnp.sum(e)
        alpha = e / s
        alpha_ref[i] = alpha

        vhat = jax.lax.dot_general(
            alpha, vals, (((1,), (0,)), ((), ())),
            preferred_element_type=jnp.float32)

        a_p = jnp.sum(jnp.where(is_p, alpha, 0.0))
        old_val = vals_ref[i, pl.ds(p, 1), :]
        v_row = v_ref[pl.ds(b, 1), :]
        vhat_ref[i] = vhat + a_p * (v_row - old_val)


import functools
from jax import lax
from jax.experimental.pallas import tpu_sc as plsc

_NW = 32
_CHUNK = 16384
_NBUF = 4


def _sc_stream_probe(keys_flat_hbm, out_hbm, buf, acc, sems):
    wid = lax.axis_index("s") * 2 + lax.axis_index("c")
    per_w = (_B * _SLOTS * _DK) // _NW
    base = wid * per_w
    nchunks = per_w // _CHUNK
    acc[...] = jnp.zeros((16,), jnp.float32)

    def dma(i, slot):
        return pltpu.make_async_copy(
            keys_flat_hbm.at[pl.ds(base + i * _CHUNK, _CHUNK)],
            buf.at[slot], sems.at[slot])

    for i in range(_NBUF):
        dma(i, i).start()
    for i in range(nchunks):
        slot = i % _NBUF
        dma(i, slot).wait()
        acc[...] = acc[...] + buf[slot, pl.ds(0, 16)]
        nxt = i + _NBUF
        if nxt < nchunks:
            dma(nxt, slot).start()
    pltpu.sync_copy(acc, out_hbm.at[wid])


def _sc_probe_call(keys):
    mesh = plsc.VectorSubcoreMesh(core_axis_name="c", subcore_axis_name="s")
    fn = functools.partial(
        pl.kernel, mesh=mesh,
        out_type=jax.ShapeDtypeStruct((_NW, 16), jnp.float32),
        scratch_types=[
            pltpu.VMEM((_NBUF, _CHUNK), jnp.float32),
            pltpu.VMEM((16,), jnp.float32),
            pltpu.SemaphoreType.DMA((_NBUF,)),
        ],
    )(_sc_stream_probe)
    return fn(keys.reshape(_B * _SLOTS * _DK))


def kernel(keys, vals, ptr, k, v, c):
    vhat, alpha = pl.pallas_call(
        _recall_body,
        grid=(_B // _BPB,),
        in_specs=[
            pl.BlockSpec(memory_space=pltpu.SMEM),
            pl.BlockSpec((_B, _DK), lambda g: (0, 0)),
            pl.BlockSpec((_B, _DV), lambda g: (0, 0)),
            pl.BlockSpec((_B, _DK), lambda g: (0, 0)),
            pl.BlockSpec((_BPB, _SLOTS, _DK), lambda g: (g, 0, 0)),
            pl.BlockSpec((_BPB, _SLOTS, _DV), lambda g: (g, 0, 0)),
        ],
        out_specs=[
            pl.BlockSpec((_BPB, 1, _DV), lambda g: (g, 0, 0)),
            pl.BlockSpec((_BPB, 1, _SLOTS), lambda g: (g, 0, 0)),
        ],
        out_shape=[
            jax.ShapeDtypeStruct((_B, 1, _DV), jnp.float32),
            jax.ShapeDtypeStruct((_B, 1, _SLOTS), jnp.float32),
        ],
    )(ptr, k, v, c, keys, vals)
    probe = _sc_probe_call(keys)
    vhat2 = vhat.reshape(_B, _DV) + probe.sum() * 1e-30
    return (vhat2, alpha.reshape(_B, _SLOTS))


# speedup vs baseline: 1.3571x; 1.3571x over previous
import jax
import jax.numpy as jnp
from jax.experimental import pallas as pl
from jax.experimental.pallas import tpu as pltpu

_B = 64
_SLOTS = 4096
_DK = 128
_DV = 128
_BPB = 4


def _recall_body(ptr_ref, k_ref, v_ref, c_ref, keys_ref, vals_ref,
                 vhat_ref, alpha_ref):
    g = pl.program_id(0)
    col = jax.lax.broadcasted_iota(jnp.int32, (1, _SLOTS), 1)
    for i in range(_BPB):
        b = g * _BPB + i
        p = ptr_ref[b]
        keys = keys_ref[i]
        vals = vals_ref[i]
        c_row = c_ref[pl.ds(b, 1), :]

        sims = jax.lax.dot_general(
            c_row, keys, (((1,), (1,)), ((), ())),
            preferred_element_type=jnp.float32)

        d_new = jnp.sum(k_ref[pl.ds(b, 1), :] * c_row)
        is_p = col == p
        sims = jnp.where(is_p, d_new, sims)

        m = jnp.max(sims)
        e = jnp.exp(sims - m)
        s = jnp.sum(e)
        alpha = e / s
        alpha_ref[i] = alpha

        vhat = jax.lax.dot_general(
            alpha, vals, (((1,), (0,)), ((), ())),
            preferred_element_type=jnp.float32)

        a_p = jnp.sum(jnp.where(is_p, alpha, 0.0))
        old_val = vals_ref[i, pl.ds(p, 1), :]
        v_row = v_ref[pl.ds(b, 1), :]
        vhat_ref[i] = vhat + a_p * (v_row - old_val)


import functools
from jax import lax
from jax.experimental.pallas import tpu_sc as plsc

_NW = 32
_CHUNK = 16384
_NBUF = 4


def _sc_stream_probe(keys_flat_hbm, out_hbm, buf, acc, sems):
    wid = lax.axis_index("s") * 2 + lax.axis_index("c")
    per_w = (_B * _SLOTS * _DK) // _NW
    base = wid * per_w
    nchunks = 1
    acc[...] = jnp.zeros((16,), jnp.float32)

    def dma(i, slot):
        return pltpu.make_async_copy(
            keys_flat_hbm.at[pl.ds(base + i * _CHUNK, _CHUNK)],
            buf.at[slot], sems.at[slot])

    for i in range(_NBUF):
        dma(i, i).start()
    for i in range(nchunks):
        slot = i % _NBUF
        dma(i, slot).wait()
        acc[...] = acc[...] + buf[slot, pl.ds(0, 16)]
        nxt = i + _NBUF
        if nxt < nchunks:
            dma(nxt, slot).start()
    pltpu.sync_copy(acc, out_hbm.at[wid])


def _sc_probe_call(keys):
    mesh = plsc.VectorSubcoreMesh(core_axis_name="c", subcore_axis_name="s")
    fn = functools.partial(
        pl.kernel, mesh=mesh,
        out_type=jax.ShapeDtypeStruct((_NW, 16), jnp.float32),
        scratch_types=[
            pltpu.VMEM((_NBUF, _CHUNK), jnp.float32),
            pltpu.VMEM((16,), jnp.float32),
            pltpu.SemaphoreType.DMA((_NBUF,)),
        ],
    )(_sc_stream_probe)
    return fn(keys.reshape(_B * _SLOTS * _DK))


def kernel(keys, vals, ptr, k, v, c):
    vhat, alpha = pl.pallas_call(
        _recall_body,
        grid=(_B // _BPB,),
        in_specs=[
            pl.BlockSpec(memory_space=pltpu.SMEM),
            pl.BlockSpec((_B, _DK), lambda g: (0, 0)),
            pl.BlockSpec((_B, _DV), lambda g: (0, 0)),
            pl.BlockSpec((_B, _DK), lambda g: (0, 0)),
            pl.BlockSpec((_BPB, _SLOTS, _DK), lambda g: (g, 0, 0)),
            pl.BlockSpec((_BPB, _SLOTS, _DV), lambda g: (g, 0, 0)),
        ],
        out_specs=[
            pl.BlockSpec((_BPB, 1, _DV), lambda g: (g, 0, 0)),
            pl.BlockSpec((_BPB, 1, _SLOTS), lambda g: (g, 0, 0)),
        ],
        out_shape=[
            jax.ShapeDtypeStruct((_B, 1, _DV), jnp.float32),
            jax.ShapeDtypeStruct((_B, 1, _SLOTS), jnp.float32),
        ],
    )(ptr, k, v, c, keys, vals)
    probe = _sc_probe_call(keys)
    vhat2 = vhat.reshape(_B, _DV) + probe.sum() * 1e-30
    return (vhat2, alpha.reshape(_B, _SLOTS))


# speedup vs baseline: 1.6417x; 1.2097x over previous
import jax
import jax.numpy as jnp
from jax.experimental import pallas as pl
from jax.experimental.pallas import tpu as pltpu

_B = 64
_SLOTS = 4096
_DK = 128
_DV = 128
_BPB = 4


def _recall_body(ptr_ref, k_ref, v_ref, c_ref, keys_ref, vals_ref,
                 vhat_ref, alpha_ref):
    g = pl.program_id(0)
    col = jax.lax.broadcasted_iota(jnp.int32, (1, _SLOTS), 1)
    for i in range(_BPB):
        b = g * _BPB + i
        p = ptr_ref[b]
        keys = keys_ref[i]
        vals = vals_ref[i]
        c_row = c_ref[pl.ds(b, 1), :]

        sims = jax.lax.dot_general(
            c_row, keys, (((1,), (1,)), ((), ())),
            preferred_element_type=jnp.float32)

        d_new = jnp.sum(k_ref[pl.ds(b, 1), :] * c_row)
        is_p = col == p
        sims = jnp.where(is_p, d_new, sims)

        m = jnp.max(sims)
        e = jnp.exp(sims - m)
        s = jnp.sum(e)
        alpha = e / s
        alpha_ref[i] = alpha

        vhat = jax.lax.dot_general(
            alpha, vals, (((1,), (0,)), ((), ())),
            preferred_element_type=jnp.float32)

        a_p = jnp.sum(jnp.where(is_p, alpha, 0.0))
        old_val = vals_ref[i, pl.ds(p, 1), :]
        v_row = v_ref[pl.ds(b, 1), :]
        vhat_ref[i] = vhat + a_p * (v_row - old_val)


def kernel(keys, vals, ptr, k, v, c):
    vhat, alpha = pl.pallas_call(
        _recall_body,
        grid=(_B // _BPB,),
        in_specs=[
            pl.BlockSpec(memory_space=pltpu.SMEM),
            pl.BlockSpec((_B, _DK), lambda g: (0, 0)),
            pl.BlockSpec((_B, _DV), lambda g: (0, 0)),
            pl.BlockSpec((_B, _DK), lambda g: (0, 0)),
            pl.BlockSpec((_BPB, _SLOTS, _DK), lambda g: (g, 0, 0)),
            pl.BlockSpec((_BPB, _SLOTS, _DV), lambda g: (g, 0, 0)),
        ],
        out_specs=[
            pl.BlockSpec((_BPB, 1, _DV), lambda g: (g, 0, 0)),
            pl.BlockSpec((_BPB, 1, _SLOTS), lambda g: (g, 0, 0)),
        ],
        out_shape=[
            jax.ShapeDtypeStruct((_B, 1, _DV), jnp.float32),
            jax.ShapeDtypeStruct((_B, 1, _SLOTS), jnp.float32),
        ],
    )(ptr, k, v, c, keys, vals)
    return (vhat.reshape(_B, _DV), alpha.reshape(_B, _SLOTS))
